# Initial kernel scaffold; baseline (speedup 1.0000x reference)
import jax
import jax.numpy as jnp
from jax.experimental import pallas as pl


def kernel(x):
    raise NotImplementedError("write your pallas kernel here")


# speedup vs baseline: 12.2826x; 12.2826x over previous
import jax
import jax.numpy as jnp
from jax.experimental import pallas as pl
from jax.experimental.pallas import tpu as pltpu

_OUT = 200
_IOU_T = 0.5
_SCORE_T = 0.01
_NC = 1024
_NEG = -1e9
_BLK = 128
_NB = _NC // _BLK
_B = 32
_NIMG = 4
_NCLS = 8
_PAD = 256


def _iou_block(cT, c):
    x1j, y1j, x2j, y2j, aj = [t[:, :, None] for t in cT]
    x1l, y1l, x2l, y2l, al = [t[None, :, :] for t in c]
    iw = jnp.maximum(jnp.minimum(x2j, x2l) - jnp.maximum(x1j, x1l), 0.0)
    ih = jnp.maximum(jnp.minimum(y2j, y2l) - jnp.maximum(y1j, y1l), 0.0)
    inter = iw * ih
    union = (aj + al) - inter
    return inter / jnp.maximum(union, 1e-8)


def _ranks(ks, bs):
    Bb, N = ks.shape
    nb = N // bs
    ii = jax.lax.broadcasted_iota(jnp.int32, (1, bs, bs), 1)
    jj = jax.lax.broadcasted_iota(jnp.int32, (1, bs, bs), 2)
    out = []
    for bi in range(nb):
        ksi = ks[:, bi * bs:(bi + 1) * bs][:, :, None]
        cnt = jnp.zeros((Bb, bs), jnp.float32)
        for bj in range(nb):
            ksj = ks[:, bj * bs:(bj + 1) * bs][:, None, :]
            gt = ksj > ksi
            if bj < bi:
                c = gt | (ksj == ksi)
            elif bj == bi:
                c = gt | ((ksj == ksi) & (jj < ii))
            else:
                c = gt
            cnt = cnt + jnp.sum(jnp.where(c, 1.0, 0.0), axis=2)
        out.append(cnt)
    return jnp.concatenate(out, axis=1)


def _nms_body(x1r, y1r, x2r, y2r, sr, outr, obbr):
    x1 = x1r[...]
    y1 = y1r[...]
    x2 = x2r[...]
    y2 = y2r[...]
    s = sr[...]
    area = (x2 - x1) * (y2 - y1)

    x1T = jnp.transpose(x1)
    y1T = jnp.transpose(y1)
    x2T = jnp.transpose(x2)
    y2T = jnp.transpose(y2)
    areaT = jnp.transpose(area)

    valid = jnp.where(s > _NEG / 2, 1.0, 0.0)

    lane = jax.lax.broadcasted_iota(jnp.int32, (1, _BLK), 1)
    jix = jax.lax.broadcasted_iota(jnp.int32, (_BLK, 1, _BLK), 0)
    lix = jax.lax.broadcasted_iota(jnp.int32, (_BLK, 1, _BLK), 2)

    pend = [jnp.ones((_B, _BLK), jnp.float32) for _ in range(_NB)]
    keeps = []
    for b in range(_NB):
        sl = slice(b * _BLK, (b + 1) * _BLK)
        cTb = (x1T[sl], y1T[sl], x2T[sl], y2T[sl], areaT[sl])
        cb = (x1[:, sl], y1[:, sl], x2[:, sl], y2[:, sl], area[:, sl])
        kb = valid[:, sl] * pend[b]

        q = _iou_block(cTb, cb)
        obbr[...] = jnp.where((q > _IOU_T) & (lix > jix), 1.0, 0.0)

        def body(j, kb_):
            row = obbr[pl.ds(j, 1), :, :][0]
            oh = jnp.where(lane == j, 1.0, 0.0)
            kj = jnp.sum(kb_ * oh, axis=1, keepdims=True)
            return kb_ * (1.0 - row * kj)

        kb = jax.lax.fori_loop(0, _BLK, body, kb)
        keeps.append(kb)

        if b < _NB - 1:
            kbT = jnp.transpose(kb)[:, :, None]
            for c in range(b + 1, _NB):
                slc = slice(c * _BLK, (c + 1) * _BLK)
                cc = (x1[:, slc], y1[:, slc], x2[:, slc], y2[:, slc],
                      area[:, slc])
                qc = _iou_block(cTb, cc)
                oc = jnp.where(qc > _IOU_T, 1.0, 0.0)
                sup = jnp.max(oc * kbT, axis=0)
                pend[c] = pend[c] * (1.0 - sup)

    keep = jnp.concatenate(keeps, axis=1)
    ks = jnp.where(keep > 0.5, s, _NEG)

    rank = _ranks(ks, 256)
    r_iota = jax.lax.broadcasted_iota(jnp.int32, (1, _PAD, 1), 1)
    r_iota_f = r_iota.astype(jnp.float32)
    cols = (ks, x1, y1, x2, y2)
    acc = [jnp.zeros((_B, _PAD), jnp.float32) for _ in range(5)]
    for jb in range(4):
        slj = slice(jb * 256, (jb + 1) * 256)
        oh = jnp.where(rank[:, slj][:, None, :] == r_iota_f, 1.0, 0.0)
        for k in range(5):
            acc[k] = acc[k] + jnp.sum(oh * cols[k][:, slj][:, None, :], axis=2)
    s256, x1c, y1c, x2c, y2c = acc

    rmask = jax.lax.broadcasted_iota(jnp.int32, (1, _PAD), 1) >= _OUT
    s256 = jnp.where(rmask, _NEG, s256)

    gs = s256.reshape(_NIMG, _NCLS * _PAD)
    gx1 = x1c.reshape(_NIMG, _NCLS * _PAD)
    gy1 = y1c.reshape(_NIMG, _NCLS * _PAD)
    gx2 = x2c.reshape(_NIMG, _NCLS * _PAD)
    gy2 = y2c.reshape(_NIMG, _NCLS * _PAD)
    gcls = (jax.lax.broadcasted_iota(jnp.int32, (_NIMG, _NCLS * _PAD), 1)
            // _PAD).astype(jnp.float32)

    grank = _ranks(gs, 256)
    gcols = (gcls, gs, gx1, gy1, gx2, gy2)
    facc = [jnp.zeros((_NIMG, _PAD), jnp.float32) for _ in range(6)]
    for jb in range(8):
        slj = slice(jb * 256, (jb + 1) * 256)
        oh = jnp.where(grank[:, slj][:, None, :] == r_iota_f, 1.0, 0.0)
        for k in range(6):
            facc[k] = facc[k] + jnp.sum(
                oh * gcols[k][:, slj][:, None, :], axis=2)
    fcls, fs, fx1, fy1, fx2, fy2 = facc

    fvalid = fs > _NEG / 2
    fx1 = jnp.clip(fx1, 0.0, 1.0)
    fy1 = jnp.clip(fy1, 0.0, 1.0)
    fx2 = jnp.clip(fx2, 0.0, 1.0)
    fy2 = jnp.clip(fy2, 0.0, 1.0)
    outc = (fcls, fs, fx1, fy1, fx2, fy2)
    col = jax.lax.broadcasted_iota(jnp.int32, (_NIMG, _PAD, 8), 2)
    vals = jnp.zeros((_NIMG, _PAD, 8), jnp.float32)
    for k in range(6):
        vals = vals + jnp.where(col == k, outc[k][:, :, None], 0.0)
    vals = jnp.where(fvalid[:, :, None], vals, 0.0)
    outr[...] = vals


def kernel(x):
    boxes = x[:, :, :4]
    scores = x[:, :, 4:]
    s = jnp.where(scores > _SCORE_T, scores, _NEG)
    st = jnp.transpose(s, (0, 2, 1)).reshape(_B, 20000)
    top_s, top_i = jax.lax.top_k(st, _NC)
    bfull = jnp.broadcast_to(
        boxes[:, None], (_NIMG, _NCLS, 20000, 4)).reshape(_B, 20000, 4)
    top_b = jnp.take_along_axis(bfull, top_i[:, :, None], axis=1)
    x1 = top_b[:, :, 0]
    y1 = top_b[:, :, 1]
    x2 = top_b[:, :, 2]
    y2 = top_b[:, :, 3]
    out = pl.pallas_call(
        _nms_body,
        out_shape=jax.ShapeDtypeStruct((_NIMG, _PAD, 8), jnp.float32),
        scratch_shapes=[pltpu.VMEM((_BLK, _B, _BLK), jnp.float32)],
    )(x1, y1, x2, y2, top_s)
    return out[:, :_OUT, :6]
